# Initial kernel scaffold; baseline (speedup 1.0000x reference)
import jax
import jax.numpy as jnp
from jax.experimental import pallas as pl


def kernel(x, edge_index, edge_attr, node_batch, overall, edge_type, W_rel, W_root, b_conv, W_edge, b_edge, W_fc1, b_fc1, W_fc2, b_fc2):
    raise NotImplementedError("write your pallas kernel here")


# speedup vs baseline: 8.5457x; 8.5457x over previous
import functools

import jax
import jax.numpy as jnp
from jax import lax
from jax.experimental import pallas as pl
from jax.experimental.pallas import tpu as pltpu
from jax.experimental.pallas import tpu_sc as plsc

N = 10000
E = 320000
F_IN = 128
H = 128
L = 3
R = 4
D_E = 16
G = 64

NC = 2
NS = 16
NW = NC * NS
C = 80
N_PAD = 10240


def _gates_body(attr_ref, w_ref, b_ref, out_ref):
    a = attr_ref[...]
    g = jnp.dot(a, w_ref[0], preferred_element_type=jnp.float32) + b_ref[0]
    out_ref[0] = jnp.maximum(g, 0.0)


def _gates_call(edge_attr, W_edge, b_edge):
    BE = 4000
    return pl.pallas_call(
        _gates_body,
        grid=(L, E // BE),
        in_specs=[
            pl.BlockSpec((BE, D_E), lambda l, i: (i, 0)),
            pl.BlockSpec((1, D_E, H), lambda l, i: (l, 0, 0)),
            pl.BlockSpec((1, 1, H), lambda l, i: (l, 0, 0)),
        ],
        out_specs=pl.BlockSpec((1, BE, H), lambda l, i: (l, i, 0)),
        out_shape=jax.ShapeDtypeStruct((L, E, H), jnp.float32),
    )(edge_attr, W_edge, b_edge)


def _prep0_body(x_ref, w2_ref, wr_ref, bc_ref, table_ref, hroot_ref):
    h = x_ref[...]
    table_ref[...] = jnp.dot(h, w2_ref[...], preferred_element_type=jnp.float32)
    hroot_ref[...] = (
        jnp.dot(h, wr_ref[...], preferred_element_type=jnp.float32) + bc_ref[...]
    )


def _prep_body(p_ref, hprev_ref, w2_ref, wr_ref, bc_ref, table_ref, hroot_ref):
    h = jnp.maximum(p_ref[0] + p_ref[1] + hprev_ref[...], 0.0)
    table_ref[...] = jnp.dot(h, w2_ref[...], preferred_element_type=jnp.float32)
    hroot_ref[...] = (
        jnp.dot(h, wr_ref[...], preferred_element_type=jnp.float32) + bc_ref[...]
    )


_BN = 2000


def _prep0_call(x, W2l, Wrl, bcl):
    return pl.pallas_call(
        _prep0_body,
        grid=(N // _BN,),
        in_specs=[
            pl.BlockSpec((_BN, F_IN), lambda i: (i, 0)),
            pl.BlockSpec((F_IN, R * H), lambda i: (0, 0)),
            pl.BlockSpec((F_IN, H), lambda i: (0, 0)),
            pl.BlockSpec((1, H), lambda i: (0, 0)),
        ],
        out_specs=[
            pl.BlockSpec((_BN, R * H), lambda i: (i, 0)),
            pl.BlockSpec((_BN, H), lambda i: (i, 0)),
        ],
        out_shape=[
            jax.ShapeDtypeStruct((N, R * H), jnp.float32),
            jax.ShapeDtypeStruct((N, H), jnp.float32),
        ],
    )(x, W2l, Wrl, bcl)


def _prep_call(p, hroot_prev, W2l, Wrl, bcl):
    return pl.pallas_call(
        _prep_body,
        grid=(N // _BN,),
        in_specs=[
            pl.BlockSpec((2, _BN, F_IN), lambda i: (0, i, 0)),
            pl.BlockSpec((_BN, F_IN), lambda i: (i, 0)),
            pl.BlockSpec((F_IN, R * H), lambda i: (0, 0)),
            pl.BlockSpec((F_IN, H), lambda i: (0, 0)),
            pl.BlockSpec((1, H), lambda i: (0, 0)),
        ],
        out_specs=[
            pl.BlockSpec((_BN, R * H), lambda i: (i, 0)),
            pl.BlockSpec((_BN, H), lambda i: (i, 0)),
        ],
        out_shape=[
            jax.ShapeDtypeStruct((N, R * H), jnp.float32),
            jax.ShapeDtypeStruct((N, H), jnp.float32),
        ],
    )(p, hroot_prev, W2l, Wrl, bcl)


def _final_body(p_ref, hr_ref, oh_ref, w1_ref, b1_ref, w2_ref, b2_ref,
                out_ref, acc_ref):
    i = pl.program_id(0)

    @pl.when(i == 0)
    def _init():
        acc_ref[...] = jnp.zeros_like(acc_ref)

    h3 = p_ref[0] + p_ref[1] + hr_ref[...]
    acc_ref[...] += lax.dot_general(
        oh_ref[...], h3, (((0,), (0,)), ((), ())),
        preferred_element_type=jnp.float32)

    @pl.when(i == pl.num_programs(0) - 1)
    def _head():
        z = jnp.dot(acc_ref[...], w1_ref[...], preferred_element_type=jnp.float32)
        z = jnp.maximum(z + b1_ref[...], 0.0)
        out_ref[...] = (
            jnp.dot(z, w2_ref[...], preferred_element_type=jnp.float32) + b2_ref[...]
        )


def _final_call(p, hroot, onehot, W_fc1, b_fc1, W_fc2, b_fc2):
    return pl.pallas_call(
        _final_body,
        grid=(N // _BN,),
        in_specs=[
            pl.BlockSpec((2, _BN, H), lambda i: (0, i, 0)),
            pl.BlockSpec((_BN, H), lambda i: (i, 0)),
            pl.BlockSpec((_BN, G), lambda i: (i, 0)),
            pl.BlockSpec((H, H // 2), lambda i: (0, 0)),
            pl.BlockSpec((1, H // 2), lambda i: (0, 0)),
            pl.BlockSpec((H // 2, 1), lambda i: (0, 0)),
            pl.BlockSpec((1, 1), lambda i: (0, 0)),
        ],
        out_specs=pl.BlockSpec((G, 1), lambda i: (0, 0)),
        out_shape=jax.ShapeDtypeStruct((G, 1), jnp.float32),
        scratch_shapes=[pltpu.VMEM((G, H), jnp.float32)],
    )(p, hroot, onehot, W_fc1, b_fc1, W_fc2, b_fc2)


def _make_edge_call(layer):
    mesh = plsc.VectorSubcoreMesh(core_axis_name="c", subcore_axis_name="s",
                                  num_cores=NC, num_subcores=NS)
    per_w = E // NW
    n_chunks = per_w // C
    stripe = N_PAD // NS

    @functools.partial(
        pl.kernel,
        out_type=jax.ShapeDtypeStruct((NC, N_PAD, H), jnp.float32),
        mesh=mesh,
        scratch_types=[
            pltpu.VMEM((C,), jnp.int32),
            pltpu.VMEM((C,), jnp.int32),
            pltpu.VMEM((C, H), jnp.float32),
            pltpu.VMEM((C, H), jnp.float32),
            pltpu.VMEM_SHARED((N_PAD, H), jnp.float32),
        ],
    )
    def edge_k(table, gates, cidx, didx, zeros, out, cidx_v, didx_v, rows_v,
               gate_v, acc):
        c = lax.axis_index("c")
        s = lax.axis_index("s")
        wid = c * NS + s
        row0 = pl.multiple_of(s * stripe, 8)

        pltpu.sync_copy(zeros.at[pl.ds(row0, stripe)],
                        acc.at[pl.ds(row0, stripe)])
        plsc.subcore_barrier()

        base_w = wid * per_w

        def chunk(i, carry):
            base = pl.multiple_of(base_w + i * C, 8)
            pltpu.sync_copy(cidx.at[pl.ds(base, C)], cidx_v)
            pltpu.sync_copy(didx.at[pl.ds(base, C)], didx_v)
            pltpu.sync_copy(table.at[cidx_v], rows_v)
            pltpu.sync_copy(gates.at[layer, pl.ds(base, C)], gate_v)

            def mul(e, cc):
                for j in range(H // 16):
                    sl = pl.ds(j * 16, 16)
                    rows_v[e, sl] = rows_v[e, sl] * gate_v[e, sl]
                return cc

            lax.fori_loop(0, C, mul, 0)
            pltpu.sync_copy(rows_v, acc.at[didx_v], add=True)
            return carry

        lax.fori_loop(0, n_chunks, chunk, 0)
        plsc.subcore_barrier()
        pltpu.sync_copy(acc.at[pl.ds(row0, stripe)],
                        out.at[c, pl.ds(row0, stripe)])

    return edge_k


@functools.lru_cache(maxsize=None)
def _edge_call(layer):
    return _make_edge_call(layer)


def kernel(x, edge_index, edge_attr, node_batch, overall, edge_type,
           W_rel, W_root, b_conv, W_edge, b_edge,
           W_fc1, b_fc1, W_fc2, b_fc2):
    src = edge_index[0]
    dst = edge_index[1]
    cidx = src * R + edge_type
    zeros = jnp.zeros((N_PAD, H), jnp.float32)
    onehot = (node_batch[:, None]
              == jnp.arange(G, dtype=jnp.int32)[None, :]).astype(jnp.float32)
    W2 = jnp.transpose(W_rel, (0, 2, 1, 3)).reshape(L, F_IN, R * H)

    gates = _gates_call(edge_attr, W_edge, b_edge.reshape(L, 1, H))

    table, hroot = _prep0_call(x, W2[0], W_root[0], b_conv[0].reshape(1, H))
    p = _edge_call(0)(table.reshape(N * R, H), gates, cidx, dst, zeros)
    for l in range(1, L):
        table, hroot = _prep_call(p, hroot, W2[l], W_root[l],
                                  b_conv[l].reshape(1, H))
        p = _edge_call(l)(table.reshape(N * R, H), gates, cidx, dst, zeros)

    out = _final_call(p, hroot, onehot,
                      W_fc1, b_fc1.reshape(1, H // 2),
                      W_fc2, b_fc2.reshape(1, 1))
    return jnp.squeeze(out)


# speedup vs baseline: 14.3436x; 1.6785x over previous
import functools

import jax
import jax.numpy as jnp
from jax import lax
from jax.experimental import pallas as pl
from jax.experimental.pallas import tpu as pltpu
from jax.experimental.pallas import tpu_sc as plsc

N = 10000
E = 320000
F_IN = 128
H = 128
L = 3
R = 4
D_E = 16
G = 64

NC = 2
NS = 16
NW = NC * NS
C = 80
N_PAD = 10240


def _gates_body(attr_ref, w_ref, b_ref, out_ref):
    a = attr_ref[...]
    g = jnp.dot(a, w_ref[0], preferred_element_type=jnp.float32) + b_ref[0]
    out_ref[0] = jnp.maximum(g, 0.0)


def _gates_call(edge_attr, W_edge, b_edge):
    BE = 4000
    return pl.pallas_call(
        _gates_body,
        grid=(L, E // BE),
        in_specs=[
            pl.BlockSpec((BE, D_E), lambda l, i: (i, 0)),
            pl.BlockSpec((1, D_E, H), lambda l, i: (l, 0, 0)),
            pl.BlockSpec((1, 1, H), lambda l, i: (l, 0, 0)),
        ],
        out_specs=pl.BlockSpec((1, BE, H), lambda l, i: (l, i, 0)),
        out_shape=jax.ShapeDtypeStruct((L, E, H), jnp.float32),
    )(edge_attr, W_edge, b_edge)


def _prep0_body(x_ref, w2_ref, wr_ref, bc_ref, table_ref, hroot_ref):
    h = x_ref[...]
    table_ref[...] = jnp.dot(h, w2_ref[...], preferred_element_type=jnp.float32)
    hroot_ref[...] = (
        jnp.dot(h, wr_ref[...], preferred_element_type=jnp.float32) + bc_ref[...]
    )


def _prep_body(p_ref, hprev_ref, w2_ref, wr_ref, bc_ref, table_ref, hroot_ref):
    h = jnp.maximum(p_ref[0] + p_ref[1] + hprev_ref[...], 0.0)
    table_ref[...] = jnp.dot(h, w2_ref[...], preferred_element_type=jnp.float32)
    hroot_ref[...] = (
        jnp.dot(h, wr_ref[...], preferred_element_type=jnp.float32) + bc_ref[...]
    )


_BN = 2000


def _prep0_call(x, W2l, Wrl, bcl):
    return pl.pallas_call(
        _prep0_body,
        grid=(N // _BN,),
        in_specs=[
            pl.BlockSpec((_BN, F_IN), lambda i: (i, 0)),
            pl.BlockSpec((F_IN, R * H), lambda i: (0, 0)),
            pl.BlockSpec((F_IN, H), lambda i: (0, 0)),
            pl.BlockSpec((1, H), lambda i: (0, 0)),
        ],
        out_specs=[
            pl.BlockSpec((_BN, R * H), lambda i: (i, 0)),
            pl.BlockSpec((_BN, H), lambda i: (i, 0)),
        ],
        out_shape=[
            jax.ShapeDtypeStruct((N, R * H), jnp.float32),
            jax.ShapeDtypeStruct((N, H), jnp.float32),
        ],
    )(x, W2l, Wrl, bcl)


def _prep_call(p, hroot_prev, W2l, Wrl, bcl):
    return pl.pallas_call(
        _prep_body,
        grid=(N // _BN,),
        in_specs=[
            pl.BlockSpec((2, _BN, F_IN), lambda i: (0, i, 0)),
            pl.BlockSpec((_BN, F_IN), lambda i: (i, 0)),
            pl.BlockSpec((F_IN, R * H), lambda i: (0, 0)),
            pl.BlockSpec((F_IN, H), lambda i: (0, 0)),
            pl.BlockSpec((1, H), lambda i: (0, 0)),
        ],
        out_specs=[
            pl.BlockSpec((_BN, R * H), lambda i: (i, 0)),
            pl.BlockSpec((_BN, H), lambda i: (i, 0)),
        ],
        out_shape=[
            jax.ShapeDtypeStruct((N, R * H), jnp.float32),
            jax.ShapeDtypeStruct((N, H), jnp.float32),
        ],
    )(p, hroot_prev, W2l, Wrl, bcl)


def _final_body(p_ref, hr_ref, oh_ref, w1_ref, b1_ref, w2_ref, b2_ref,
                out_ref, acc_ref):
    i = pl.program_id(0)

    @pl.when(i == 0)
    def _init():
        acc_ref[...] = jnp.zeros_like(acc_ref)

    h3 = p_ref[0] + p_ref[1] + hr_ref[...]
    acc_ref[...] += lax.dot_general(
        oh_ref[...], h3, (((0,), (0,)), ((), ())),
        preferred_element_type=jnp.float32)

    @pl.when(i == pl.num_programs(0) - 1)
    def _head():
        z = jnp.dot(acc_ref[...], w1_ref[...], preferred_element_type=jnp.float32)
        z = jnp.maximum(z + b1_ref[...], 0.0)
        out_ref[...] = (
            jnp.dot(z, w2_ref[...], preferred_element_type=jnp.float32) + b2_ref[...]
        )


def _final_call(p, hroot, onehot, W_fc1, b_fc1, W_fc2, b_fc2):
    return pl.pallas_call(
        _final_body,
        grid=(N // _BN,),
        in_specs=[
            pl.BlockSpec((2, _BN, H), lambda i: (0, i, 0)),
            pl.BlockSpec((_BN, H), lambda i: (i, 0)),
            pl.BlockSpec((_BN, G), lambda i: (i, 0)),
            pl.BlockSpec((H, H // 2), lambda i: (0, 0)),
            pl.BlockSpec((1, H // 2), lambda i: (0, 0)),
            pl.BlockSpec((H // 2, 1), lambda i: (0, 0)),
            pl.BlockSpec((1, 1), lambda i: (0, 0)),
        ],
        out_specs=pl.BlockSpec((G, 1), lambda i: (0, 0)),
        out_shape=jax.ShapeDtypeStruct((G, 1), jnp.float32),
        scratch_shapes=[pltpu.VMEM((G, H), jnp.float32)],
    )(p, hroot, onehot, W_fc1, b_fc1, W_fc2, b_fc2)


def _make_edge_call(layer):
    mesh = plsc.VectorSubcoreMesh(core_axis_name="c", subcore_axis_name="s",
                                  num_cores=NC, num_subcores=NS)
    per_w = E // NW
    n_chunks = per_w // C
    n_pairs = (n_chunks - 1) // 2
    stripe = N_PAD // NS

    @functools.partial(
        pl.kernel,
        out_type=jax.ShapeDtypeStruct((NC, N_PAD, H), jnp.float32),
        mesh=mesh,
        scratch_types=[
            pltpu.VMEM((2, C), jnp.int32),
            pltpu.VMEM((2, C), jnp.int32),
            pltpu.VMEM((C, H), jnp.float32),
            pltpu.VMEM((C, H), jnp.float32),
            pltpu.VMEM((C, H), jnp.float32),
            pltpu.VMEM((C, H), jnp.float32),
            pltpu.VMEM_SHARED((N_PAD, H), jnp.float32),
            pltpu.SemaphoreType.DMA,
            pltpu.SemaphoreType.DMA,
            pltpu.SemaphoreType.DMA,
            pltpu.SemaphoreType.DMA,
            pltpu.SemaphoreType.DMA,
            pltpu.SemaphoreType.DMA,
        ],
    )
    def edge_k(table, gates, pidx, zeros, out, idx_a, idx_b,
               rows_a, rows_b, gate_a, gate_b, acc,
               sem_ra, sem_rb, sem_ta, sem_tb, sem_sa, sem_sb):
        c = lax.axis_index("c")
        s = lax.axis_index("s")
        wid = c * NS + s
        row0 = pl.multiple_of(s * stripe, 8)

        pltpu.sync_copy(zeros.at[pl.ds(row0, stripe)],
                        acc.at[pl.ds(row0, stripe)])
        plsc.subcore_barrier()

        base_w = wid * per_w

        def start_fetch(i, idx_v, rows_v, gate_v, sem_r, sem_t):
            base = pl.multiple_of(base_w + i * C, 8)
            pltpu.sync_copy(pidx.at[wid, i], idx_v)
            pltpu.async_copy(table.at[idx_v.at[0]], rows_v, sem_r)
            pltpu.async_copy(gates.at[layer, pl.ds(base, C)], gate_v, sem_t)

        def wait_fetch(idx_v, rows_v, gate_v, sem_r, sem_t):
            pltpu.make_async_copy(table.at[idx_v.at[0]], rows_v, sem_r).wait()
            pltpu.make_async_copy(gates.at[layer, pl.ds(0, C)], gate_v,
                                  sem_t).wait()

        def mul(rows_v, gate_v):
            def body(e, cc):
                for j in range(H // 16):
                    sl = pl.ds(j * 16, 16)
                    rows_v[e, sl] = rows_v[e, sl] * gate_v[e, sl]
                return cc
            lax.fori_loop(0, C, body, 0)

        def start_scatter(idx_v, rows_v, sem_s):
            pltpu.async_copy(rows_v, acc.at[idx_v.at[1]], sem_s, add=True)

        def wait_scatter(idx_v, rows_v, sem_s):
            pltpu.make_async_copy(rows_v, acc.at[idx_v.at[1]], sem_s).wait()

        start_fetch(0, idx_a, rows_a, gate_a, sem_ra, sem_ta)

        def pair(p, carry):
            a = 2 * p
            @pl.when(p > 0)
            def _():
                wait_scatter(idx_b, rows_b, sem_sb)
            start_fetch(a + 1, idx_b, rows_b, gate_b, sem_rb, sem_tb)
            wait_fetch(idx_a, rows_a, gate_a, sem_ra, sem_ta)
            mul(rows_a, gate_a)
            start_scatter(idx_a, rows_a, sem_sa)
            wait_scatter(idx_a, rows_a, sem_sa)
            start_fetch(a + 2, idx_a, rows_a, gate_a, sem_ra, sem_ta)
            wait_fetch(idx_b, rows_b, gate_b, sem_rb, sem_tb)
            mul(rows_b, gate_b)
            start_scatter(idx_b, rows_b, sem_sb)
            return carry

        lax.fori_loop(0, n_pairs, pair, 0)

        wait_fetch(idx_a, rows_a, gate_a, sem_ra, sem_ta)
        mul(rows_a, gate_a)
        start_scatter(idx_a, rows_a, sem_sa)
        wait_scatter(idx_a, rows_a, sem_sa)
        wait_scatter(idx_b, rows_b, sem_sb)

        plsc.subcore_barrier()
        pltpu.sync_copy(acc.at[pl.ds(row0, stripe)],
                        out.at[c, pl.ds(row0, stripe)])

    return edge_k


@functools.lru_cache(maxsize=None)
def _edge_call(layer):
    return _make_edge_call(layer)


def kernel(x, edge_index, edge_attr, node_batch, overall, edge_type,
           W_rel, W_root, b_conv, W_edge, b_edge,
           W_fc1, b_fc1, W_fc2, b_fc2):
    src = edge_index[0]
    dst = edge_index[1]
    n_chunks = E // NW // C
    cidx = (src * R + edge_type).reshape(NW, n_chunks, C)
    pidx = jnp.stack([cidx, dst.reshape(NW, n_chunks, C)], axis=2)
    zeros = jnp.zeros((N_PAD, H), jnp.float32)
    onehot = (node_batch[:, None]
              == jnp.arange(G, dtype=jnp.int32)[None, :]).astype(jnp.float32)
    W2 = jnp.transpose(W_rel, (0, 2, 1, 3)).reshape(L, F_IN, R * H)

    gates = _gates_call(edge_attr, W_edge, b_edge.reshape(L, 1, H))

    table, hroot = _prep0_call(x, W2[0], W_root[0], b_conv[0].reshape(1, H))
    p = _edge_call(0)(table.reshape(N * R, H), gates, pidx, zeros)
    for l in range(1, L):
        table, hroot = _prep_call(p, hroot, W2[l], W_root[l],
                                  b_conv[l].reshape(1, H))
        p = _edge_call(l)(table.reshape(N * R, H), gates, pidx, zeros)

    out = _final_call(p, hroot, onehot,
                      W_fc1, b_fc1.reshape(1, H // 2),
                      W_fc2, b_fc2.reshape(1, 1))
    return jnp.squeeze(out)


# speedup vs baseline: 15.9650x; 1.1130x over previous
import functools

import jax
import jax.numpy as jnp
from jax import lax
from jax.experimental import pallas as pl
from jax.experimental.pallas import tpu as pltpu
from jax.experimental.pallas import tpu_sc as plsc

N = 10000
E = 320000
F_IN = 128
H = 128
L = 3
R = 4
D_E = 16
G = 64

NC = 2
NS = 16
NW = NC * NS
C = 80
N_PAD = 10240


def _gates_body(attr_ref, w_ref, b_ref, out_ref):
    a = attr_ref[...]
    g = jnp.dot(a, w_ref[...], preferred_element_type=jnp.float32) + b_ref[...]
    out_ref[...] = jnp.maximum(g, 0.0)


def _gates_call(edge_attr, W_edge_l, b_edge_l):
    BE = 4000
    return pl.pallas_call(
        _gates_body,
        grid=(E // BE,),
        in_specs=[
            pl.BlockSpec((BE, D_E), lambda i: (i, 0)),
            pl.BlockSpec((D_E, H), lambda i: (0, 0)),
            pl.BlockSpec((1, H), lambda i: (0, 0)),
        ],
        out_specs=pl.BlockSpec((BE, H), lambda i: (i, 0)),
        out_shape=jax.ShapeDtypeStruct((E, H), jnp.float32),
    )(edge_attr, W_edge_l, b_edge_l)


def _prep0_body(x_ref, w2_ref, wr_ref, bc_ref, table_ref, hroot_ref):
    h = x_ref[...]
    table_ref[...] = jnp.dot(h, w2_ref[...], preferred_element_type=jnp.float32)
    hroot_ref[...] = (
        jnp.dot(h, wr_ref[...], preferred_element_type=jnp.float32) + bc_ref[...]
    )


def _prep_body(p_ref, hprev_ref, w2_ref, wr_ref, bc_ref, table_ref, hroot_ref):
    h = jnp.maximum(p_ref[0] + p_ref[1] + hprev_ref[...], 0.0)
    table_ref[...] = jnp.dot(h, w2_ref[...], preferred_element_type=jnp.float32)
    hroot_ref[...] = (
        jnp.dot(h, wr_ref[...], preferred_element_type=jnp.float32) + bc_ref[...]
    )


_BN = 2000


def _prep0_call(x, W2l, Wrl, bcl):
    return pl.pallas_call(
        _prep0_body,
        grid=(N // _BN,),
        in_specs=[
            pl.BlockSpec((_BN, F_IN), lambda i: (i, 0)),
            pl.BlockSpec((F_IN, R * H), lambda i: (0, 0)),
            pl.BlockSpec((F_IN, H), lambda i: (0, 0)),
            pl.BlockSpec((1, H), lambda i: (0, 0)),
        ],
        out_specs=[
            pl.BlockSpec((_BN, R * H), lambda i: (i, 0)),
            pl.BlockSpec((_BN, H), lambda i: (i, 0)),
        ],
        out_shape=[
            jax.ShapeDtypeStruct((N, R * H), jnp.float32),
            jax.ShapeDtypeStruct((N, H), jnp.float32),
        ],
    )(x, W2l, Wrl, bcl)


def _prep_call(p, hroot_prev, W2l, Wrl, bcl):
    return pl.pallas_call(
        _prep_body,
        grid=(N // _BN,),
        in_specs=[
            pl.BlockSpec((2, _BN, F_IN), lambda i: (0, i, 0)),
            pl.BlockSpec((_BN, F_IN), lambda i: (i, 0)),
            pl.BlockSpec((F_IN, R * H), lambda i: (0, 0)),
            pl.BlockSpec((F_IN, H), lambda i: (0, 0)),
            pl.BlockSpec((1, H), lambda i: (0, 0)),
        ],
        out_specs=[
            pl.BlockSpec((_BN, R * H), lambda i: (i, 0)),
            pl.BlockSpec((_BN, H), lambda i: (i, 0)),
        ],
        out_shape=[
            jax.ShapeDtypeStruct((N, R * H), jnp.float32),
            jax.ShapeDtypeStruct((N, H), jnp.float32),
        ],
    )(p, hroot_prev, W2l, Wrl, bcl)


def _final_body(p_ref, hr_ref, oh_ref, w1_ref, b1_ref, w2_ref, b2_ref,
                out_ref, acc_ref):
    i = pl.program_id(0)

    @pl.when(i == 0)
    def _init():
        acc_ref[...] = jnp.zeros_like(acc_ref)

    h3 = p_ref[0] + p_ref[1] + hr_ref[...]
    acc_ref[...] += lax.dot_general(
        oh_ref[...], h3, (((0,), (0,)), ((), ())),
        preferred_element_type=jnp.float32)

    @pl.when(i == pl.num_programs(0) - 1)
    def _head():
        z = jnp.dot(acc_ref[...], w1_ref[...], preferred_element_type=jnp.float32)
        z = jnp.maximum(z + b1_ref[...], 0.0)
        out_ref[...] = (
            jnp.dot(z, w2_ref[...], preferred_element_type=jnp.float32) + b2_ref[...]
        )


def _final_call(p, hroot, onehot, W_fc1, b_fc1, W_fc2, b_fc2):
    return pl.pallas_call(
        _final_body,
        grid=(N // _BN,),
        in_specs=[
            pl.BlockSpec((2, _BN, H), lambda i: (0, i, 0)),
            pl.BlockSpec((_BN, H), lambda i: (i, 0)),
            pl.BlockSpec((_BN, G), lambda i: (i, 0)),
            pl.BlockSpec((H, H // 2), lambda i: (0, 0)),
            pl.BlockSpec((1, H // 2), lambda i: (0, 0)),
            pl.BlockSpec((H // 2, 1), lambda i: (0, 0)),
            pl.BlockSpec((1, 1), lambda i: (0, 0)),
        ],
        out_specs=pl.BlockSpec((G, 1), lambda i: (0, 0)),
        out_shape=jax.ShapeDtypeStruct((G, 1), jnp.float32),
        scratch_shapes=[pltpu.VMEM((G, H), jnp.float32)],
    )(p, hroot, onehot, W_fc1, b_fc1, W_fc2, b_fc2)


def _make_edge_call():
    mesh = plsc.VectorSubcoreMesh(core_axis_name="c", subcore_axis_name="s",
                                  num_cores=NC, num_subcores=NS)
    per_w = E // NW
    n_chunks = per_w // C
    n_pairs = (n_chunks - 1) // 2
    stripe = N_PAD // NS

    @functools.partial(
        pl.kernel,
        out_type=jax.ShapeDtypeStruct((NC, N_PAD, H), jnp.float32),
        mesh=mesh,
        scratch_types=[
            pltpu.VMEM((2, C), jnp.int32),
            pltpu.VMEM((2, C), jnp.int32),
            pltpu.VMEM((C, H), jnp.float32),
            pltpu.VMEM((C, H), jnp.float32),
            pltpu.VMEM((C, H), jnp.float32),
            pltpu.VMEM((C, H), jnp.float32),
            pltpu.VMEM_SHARED((N_PAD, H), jnp.float32),
            pltpu.SemaphoreType.DMA,
            pltpu.SemaphoreType.DMA,
            pltpu.SemaphoreType.DMA,
            pltpu.SemaphoreType.DMA,
            pltpu.SemaphoreType.DMA,
            pltpu.SemaphoreType.DMA,
        ],
    )
    def edge_k(table, gates, pidx, zeros, out, idx_a, idx_b,
               rows_a, rows_b, gate_a, gate_b, acc,
               sem_ra, sem_rb, sem_ta, sem_tb, sem_sa, sem_sb):
        c = lax.axis_index("c")
        s = lax.axis_index("s")
        wid = c * NS + s
        row0 = pl.multiple_of(s * stripe, 8)

        pltpu.sync_copy(zeros.at[pl.ds(row0, stripe)],
                        acc.at[pl.ds(row0, stripe)])
        plsc.subcore_barrier()

        base_w = wid * per_w

        def start_fetch(i, idx_v, rows_v, gate_v, sem_r, sem_t):
            base = pl.multiple_of(base_w + i * C, 8)
            pltpu.sync_copy(pidx.at[wid, i], idx_v)
            pltpu.async_copy(table.at[idx_v.at[0]], rows_v, sem_r)
            pltpu.async_copy(gates.at[pl.ds(base, C)], gate_v, sem_t)

        def wait_fetch(idx_v, rows_v, gate_v, sem_r, sem_t):
            pltpu.make_async_copy(table.at[idx_v.at[0]], rows_v, sem_r).wait()
            pltpu.make_async_copy(gates.at[pl.ds(0, C)], gate_v,
                                  sem_t).wait()

        def mul(rows_v, gate_v):
            @plsc.parallel_loop(0, C, unroll=2)
            def _body(e):
                for j in range(H // 16):
                    sl = pl.ds(j * 16, 16)
                    rows_v[e, sl] = rows_v[e, sl] * gate_v[e, sl]

        def start_scatter(idx_v, rows_v, sem_s):
            pltpu.async_copy(rows_v, acc.at[idx_v.at[1]], sem_s, add=True)

        def wait_scatter(idx_v, rows_v, sem_s):
            pltpu.make_async_copy(rows_v, acc.at[idx_v.at[1]], sem_s).wait()

        start_fetch(0, idx_a, rows_a, gate_a, sem_ra, sem_ta)

        def pair(p, carry):
            a = 2 * p
            @pl.when(p > 0)
            def _():
                wait_scatter(idx_b, rows_b, sem_sb)
            start_fetch(a + 1, idx_b, rows_b, gate_b, sem_rb, sem_tb)
            wait_fetch(idx_a, rows_a, gate_a, sem_ra, sem_ta)
            mul(rows_a, gate_a)
            start_scatter(idx_a, rows_a, sem_sa)
            wait_scatter(idx_a, rows_a, sem_sa)
            start_fetch(a + 2, idx_a, rows_a, gate_a, sem_ra, sem_ta)
            wait_fetch(idx_b, rows_b, gate_b, sem_rb, sem_tb)
            mul(rows_b, gate_b)
            start_scatter(idx_b, rows_b, sem_sb)
            return carry

        lax.fori_loop(0, n_pairs, pair, 0)

        wait_fetch(idx_a, rows_a, gate_a, sem_ra, sem_ta)
        mul(rows_a, gate_a)
        start_scatter(idx_a, rows_a, sem_sa)
        wait_scatter(idx_a, rows_a, sem_sa)
        wait_scatter(idx_b, rows_b, sem_sb)

        plsc.subcore_barrier()
        pltpu.sync_copy(acc.at[pl.ds(row0, stripe)],
                        out.at[c, pl.ds(row0, stripe)])

    return edge_k


@functools.lru_cache(maxsize=None)
def _edge_call():
    return _make_edge_call()


def kernel(x, edge_index, edge_attr, node_batch, overall, edge_type,
           W_rel, W_root, b_conv, W_edge, b_edge,
           W_fc1, b_fc1, W_fc2, b_fc2):
    src = edge_index[0]
    dst = edge_index[1]
    n_chunks = E // NW // C
    cidx = (src * R + edge_type).reshape(NW, n_chunks, C)
    pidx = jnp.stack([cidx, dst.reshape(NW, n_chunks, C)], axis=2)
    zeros = jnp.zeros((N_PAD, H), jnp.float32)
    onehot = (node_batch[:, None]
              == jnp.arange(G, dtype=jnp.int32)[None, :]).astype(jnp.float32)
    W2 = jnp.transpose(W_rel, (0, 2, 1, 3)).reshape(L, F_IN, R * H)

    gates = [_gates_call(edge_attr, W_edge[l], b_edge[l].reshape(1, H))
             for l in range(L)]

    table, hroot = _prep0_call(x, W2[0], W_root[0], b_conv[0].reshape(1, H))
    p = _edge_call()(table.reshape(N * R, H), gates[0], pidx, zeros)
    for l in range(1, L):
        table, hroot = _prep_call(p, hroot, W2[l], W_root[l],
                                  b_conv[l].reshape(1, H))
        p = _edge_call()(table.reshape(N * R, H), gates[l], pidx, zeros)

    out = _final_call(p, hroot, onehot,
                      W_fc1, b_fc1.reshape(1, H // 2),
                      W_fc2, b_fc2.reshape(1, 1))
    return jnp.squeeze(out)
